# Initial kernel scaffold; baseline (speedup 1.0000x reference)
import functools

import jax
import jax.numpy as jnp
from jax import lax
from jax.experimental import pallas as pl
from jax.experimental.pallas import tpu as pltpu
from jax.experimental.pallas import tpu_sc as plsc

N_NODES = 10000
N_EDGES = 320000
NC = 2
NS = 16
L = 16

K_BATCH = 128
_NB = -(-N_EDGES // (NS * K_BATCH))
EPT = _NB * K_BATCH
E_PAD = EPT * NS
ROWS_PT = N_NODES // NS


def _mm_body(x_ref, w_ref, o_ref):
    o_ref[...] = jnp.dot(x_ref[...], w_ref[...],
                         preferred_element_type=jnp.float32)


def _mm1(x, W):
    M, K = x.shape
    _, C = W.shape
    blk = 2000
    return pl.pallas_call(
        _mm_body,
        grid=(M // blk,),
        in_specs=[pl.BlockSpec((blk, K), lambda i: (i, 0)),
                  pl.BlockSpec((K, C), lambda i: (0, 0))],
        out_specs=pl.BlockSpec((blk, C), lambda i: (i, 0)),
        out_shape=jax.ShapeDtypeStruct((M, C), jnp.float32),
    )(x, W)


def _mm2_body(a_ref, b_ref, w_ref, o_ref):
    h0 = jax.nn.relu(a_ref[0] + b_ref[0])
    h1 = jax.nn.relu(a_ref[1] + b_ref[1])
    o_ref[...] = (jnp.dot(h0, w_ref[0], preferred_element_type=jnp.float32) +
                  jnp.dot(h1, w_ref[1], preferred_element_type=jnp.float32))


def _mm2(agg, b, W):
    _, M, Cc = agg.shape
    _, _, C = W.shape
    blk = 2000
    return pl.pallas_call(
        _mm2_body,
        grid=(M // blk,),
        in_specs=[pl.BlockSpec((2, blk, Cc), lambda i: (0, i, 0)),
                  pl.BlockSpec((2, Cc), lambda i: (0, 0)),
                  pl.BlockSpec((2, Cc, C), lambda i: (0, 0, 0))],
        out_specs=pl.BlockSpec((blk, C), lambda i: (i, 0)),
        out_shape=jax.ShapeDtypeStruct((M, C), jnp.float32),
    )(agg, b, W)


def _final_body(a_ref, b_ref, o_ref):
    h0 = jax.nn.relu(a_ref[0] + b_ref[0])
    h1 = jax.nn.relu(a_ref[1] + b_ref[1])
    o_ref[...] = jnp.concatenate([h0, h1], axis=-1)


def _final(agg, b):
    _, M, Cc = agg.shape
    blk = 2000
    return pl.pallas_call(
        _final_body,
        grid=(M // blk,),
        in_specs=[pl.BlockSpec((2, blk, Cc), lambda i: (0, i, 0)),
                  pl.BlockSpec((2, Cc), lambda i: (0, 0))],
        out_specs=pl.BlockSpec((blk, 2 * Cc), lambda i: (i, 0)),
        out_shape=jax.ShapeDtypeStruct((M, 2 * Cc), jnp.float32),
    )(agg, b)


def _make_spmm(Cc):
    K = K_BATCH
    mesh = plsc.VectorSubcoreMesh(core_axis_name="c", subcore_axis_name="s")

    @functools.partial(
        pl.kernel,
        out_type=jax.ShapeDtypeStruct((NC, N_NODES, Cc), jnp.float32),
        mesh=mesh,
        scratch_types=[
            pltpu.VMEM((K,), jnp.int32),
            pltpu.VMEM((K,), jnp.int32),
            pltpu.VMEM((K,), jnp.int32),
            pltpu.VMEM((K,), jnp.float32),
            pltpu.VMEM((K, Cc), jnp.float32),
            pltpu.VMEM_SHARED((N_NODES, Cc), jnp.float32),
            pltpu.SemaphoreType.DMA,
        ],
    )
    def spmm(table, rowi, coli, ew, out, colb, idxb, rowb, wb, rowsb, acc,
             sem):
        c = lax.axis_index("c")
        s = lax.axis_index("s")

        zero = jnp.zeros((L,), jnp.float32)

        def zfill(e, carry):
            for cc in range(Cc // L):
                rowsb[e, pl.ds(cc * L, L)] = zero
            return carry

        lax.fori_loop(0, K, zfill, 0)
        for kk in range(ROWS_PT // 125):
            pltpu.sync_copy(rowsb.at[pl.ds(0, 125)],
                            acc.at[pl.ds(s * ROWS_PT + kk * 125, 125)])
        plsc.subcore_barrier()

        def batch(bi, carry):
            base = s * EPT + bi * K
            pltpu.sync_copy(coli.at[pl.ds(base, K)], colb)
            pltpu.sync_copy(rowi.at[pl.ds(base, K)], rowb)
            pltpu.sync_copy(ew.at[pl.ds(base, K)], wb)
            for j in range(K // L):
                idxb[pl.ds(j * L, L)] = colb[pl.ds(j * L, L)] * 2 + c
            pltpu.async_copy(table.at[idxb], rowsb, sem).wait()

            def scale(j, inner):
                for jj in range(L):
                    e = j * L + jj
                    wv = plsc.load_gather(
                        wb, [jnp.full((L,), e, jnp.int32)])
                    for cc in range(Cc // L):
                        sl = pl.ds(cc * L, L)
                        rowsb[e, sl] = rowsb[e, sl] * wv
                return inner

            lax.fori_loop(0, K // L, scale, 0)
            pltpu.sync_copy(rowsb, acc.at[rowb], add=True)
            return carry

        lax.fori_loop(0, _NB, batch, 0)
        plsc.subcore_barrier()

        for kk in range(ROWS_PT // 125):
            r0 = s * ROWS_PT + kk * 125
            pltpu.sync_copy(acc.at[pl.ds(r0, 125)], out.at[c, pl.ds(r0, 125)])

    return spmm


_spmm128 = _make_spmm(128)
_spmm64 = _make_spmm(64)


@jax.jit
def kernel(x, edge_index, edge_weight, W1, b1, W2, b2):
    row = edge_index[0]
    col = edge_index[1]
    pad = E_PAD - N_EDGES
    rowp = jnp.pad(row, (0, pad))
    colp = jnp.pad(col, (0, pad))
    ewp = jnp.pad(edge_weight, (0, pad))

    s1 = _mm1(x, W1)
    agg1 = _spmm128(s1.reshape(2 * N_NODES, 128), rowp, colp, ewp)
    s2 = _mm2(agg1, b1.reshape(2, 128), W2.reshape(2, 128, 128))
    agg2 = _spmm64(s2.reshape(2 * N_NODES, 64), rowp, colp, ewp)
    return _final(agg2, b2.reshape(2, 64))


# speedup vs baseline: 3.1867x; 3.1867x over previous
import functools

import jax
import jax.numpy as jnp
from jax import lax
from jax.experimental import pallas as pl
from jax.experimental.pallas import tpu as pltpu
from jax.experimental.pallas import tpu_sc as plsc

N_NODES = 10000
N_EDGES = 320000
NC = 2
NS = 16
L = 16

K_BATCH = 128
E_PAD = NC * NS * K_BATCH * (-(-N_EDGES // (NC * NS * K_BATCH)))
NP = 10240
ROWS_PT = NP // NS


def _mm_body(x_ref, w_ref, o_ref):
    o_ref[...] = jnp.dot(x_ref[...], w_ref[...],
                         preferred_element_type=jnp.float32)


def _mm1(x, W):
    M, K = x.shape
    _, C = W.shape
    blk = 2000
    return pl.pallas_call(
        _mm_body,
        grid=(M // blk,),
        in_specs=[pl.BlockSpec((blk, K), lambda i: (i, 0)),
                  pl.BlockSpec((K, C), lambda i: (0, 0))],
        out_specs=pl.BlockSpec((blk, C), lambda i: (i, 0)),
        out_shape=jax.ShapeDtypeStruct((M, C), jnp.float32),
    )(x, W)


def _mm2_body(a_ref, b_ref, w_ref, o_ref):
    h0 = jax.nn.relu(a_ref[0] + b_ref[0])
    h1 = jax.nn.relu(a_ref[1] + b_ref[1])
    o_ref[...] = (jnp.dot(h0, w_ref[0], preferred_element_type=jnp.float32) +
                  jnp.dot(h1, w_ref[1], preferred_element_type=jnp.float32))


def _mm2(agg, b, W):
    _, _, Cc = agg.shape
    _, _, C = W.shape
    M = N_NODES
    blk = 2000
    return pl.pallas_call(
        _mm2_body,
        grid=(M // blk,),
        in_specs=[pl.BlockSpec((2, blk, Cc), lambda i: (0, i, 0)),
                  pl.BlockSpec((2, Cc), lambda i: (0, 0)),
                  pl.BlockSpec((2, Cc, C), lambda i: (0, 0, 0))],
        out_specs=pl.BlockSpec((blk, C), lambda i: (i, 0)),
        out_shape=jax.ShapeDtypeStruct((M, C), jnp.float32),
    )(agg, b, W)


def _final_body(a_ref, b_ref, o_ref):
    o_ref[...] = jax.nn.relu(a_ref[0] + a_ref[1] + b_ref[...])


def _final(agg, b):
    _, _, C = agg.shape
    M = N_NODES
    blk = 2000
    return pl.pallas_call(
        _final_body,
        grid=(M // blk,),
        in_specs=[pl.BlockSpec((2, blk, C), lambda i: (0, i, 0)),
                  pl.BlockSpec((1, C), lambda i: (0, 0))],
        out_specs=pl.BlockSpec((blk, C), lambda i: (i, 0)),
        out_shape=jax.ShapeDtypeStruct((M, C), jnp.float32),
    )(agg, b)


def _make_spmm(chunked):
    Cc = 128
    K = K_BATCH
    n_workers = NS if chunked else NC * NS
    ept = E_PAD // n_workers
    nb = ept // K
    mesh = plsc.VectorSubcoreMesh(core_axis_name="c", subcore_axis_name="s")

    @functools.partial(
        pl.kernel,
        out_type=jax.ShapeDtypeStruct((NC, NP, Cc), jnp.float32),
        mesh=mesh,
        scratch_types=[
            pltpu.VMEM((K,), jnp.int32),
            pltpu.VMEM((K,), jnp.int32),
            pltpu.VMEM((K,), jnp.int32),
            pltpu.VMEM((K,), jnp.float32),
            pltpu.VMEM((K, Cc), jnp.float32),
            pltpu.VMEM_SHARED((NP, Cc), jnp.float32),
            pltpu.SemaphoreType.DMA,
        ],
    )
    def spmm(table, rowi, coli, ew, out, colb, idxb, rowb, wb, rowsb, acc,
             sem):
        c = lax.axis_index("c")
        s = lax.axis_index("s")

        zero = jnp.zeros((L,), jnp.float32)

        def zfill(e, carry):
            for cc in range(Cc // L):
                rowsb[e, pl.ds(cc * L, L)] = zero
            return carry

        lax.fori_loop(0, K, zfill, 0)
        for kk in range(ROWS_PT // K):
            pltpu.sync_copy(rowsb,
                            acc.at[pl.ds(s * ROWS_PT + kk * K, K)])
        plsc.subcore_barrier()

        def batch(bi, carry):
            if chunked:
                base = s * ept + bi * K
            else:
                base = (c * NS + s) * ept + bi * K
            pltpu.sync_copy(coli.at[pl.ds(base, K)], colb)
            pltpu.sync_copy(rowi.at[pl.ds(base, K)], rowb)
            pltpu.sync_copy(ew.at[pl.ds(base, K)], wb)
            for j in range(K // L):
                if chunked:
                    idxb[pl.ds(j * L, L)] = colb[pl.ds(j * L, L)] * 2 + c
                else:
                    idxb[pl.ds(j * L, L)] = colb[pl.ds(j * L, L)]
            pltpu.async_copy(table.at[idxb], rowsb, sem).wait()

            def scale(j, inner):
                w16 = wb[pl.ds(j * L, L)]
                for jj in range(L):
                    e = j * L + jj
                    wv = jnp.broadcast_to(w16[jj], (L,))
                    for cc in range(Cc // L):
                        sl = pl.ds(cc * L, L)
                        rowsb[e, sl] = rowsb[e, sl] * wv
                return inner

            lax.fori_loop(0, K // L, scale, 0)
            pltpu.sync_copy(rowsb, acc.at[rowb], add=True)
            return carry

        lax.fori_loop(0, nb, batch, 0)
        plsc.subcore_barrier()

        for kk in range(ROWS_PT // K):
            r0 = s * ROWS_PT + kk * K
            pltpu.sync_copy(acc.at[pl.ds(r0, K)], out.at[c, pl.ds(r0, K)])

    return spmm


_spmm_chunked = _make_spmm(True)
_spmm_split = _make_spmm(False)


@jax.jit
def kernel(x, edge_index, edge_weight, W1, b1, W2, b2):
    row = edge_index[0]
    col = edge_index[1]
    pad = E_PAD - N_EDGES
    rowp = jnp.pad(row, (0, pad))
    colp = jnp.pad(col, (0, pad))
    ewp = jnp.pad(edge_weight, (0, pad))

    s1 = _mm1(x, W1)
    agg1 = _spmm_chunked(s1.reshape(2 * N_NODES, 128), rowp, colp, ewp)
    s2 = _mm2(agg1, b1.reshape(2, 128), W2.reshape(2, 128, 128))
    agg2 = _spmm_split(s2, rowp, colp, ewp)
    return _final(agg2, b2.reshape(1, 128))
